# baseline (speedup 1.0000x reference)
import functools

import jax
import jax.numpy as jnp
from jax import lax
from jax.experimental import pallas as pl
from jax.experimental.pallas import tpu as pltpu
from jax.experimental.pallas import tpu_sc as plsc

N = 10000
E = 320000
H = 128

NC = 2
NS = 16
NW = NC * NS
EPW = E // NW
C = 40
NCHUNK = EPW // C
NP = 10240
RPT = NP // NS
RCH = C
NRCH = RPT // RCH


def _edge_body(k_hbm, q_hbm, v_hbm, src_hbm, dst_hbm, zeros_hbm, out_hbm,
               sidx, didx, kd, qs, vs, gsems, ssems, aggsh):
    cid = lax.axis_index("c")
    sid = lax.axis_index("s")
    wid = sid * NC + cid

    row0 = sid * RPT
    for c in range(NRCH):
        pltpu.sync_copy(zeros_hbm, aggsh.at[pl.ds(row0 + c * RCH, RCH)])
    plsc.subcore_barrier()
    zbuf = kd.at[0]

    def issue_idx(i, b):
        base = wid * EPW + i * C
        pltpu.sync_copy(src_hbm.at[pl.ds(base, C)], sidx.at[b])
        pltpu.sync_copy(dst_hbm.at[pl.ds(base, C)], didx.at[b])

    def issue_gathers(b):
        pltpu.async_copy(k_hbm.at[didx.at[b]], kd.at[b], gsems[b])
        pltpu.async_copy(q_hbm.at[sidx.at[b]], qs.at[b], gsems[b])
        pltpu.async_copy(v_hbm.at[sidx.at[b]], vs.at[b], gsems[b])

    def wait_gathers(b):
        pltpu.make_async_copy(k_hbm.at[didx.at[b]], kd.at[b], gsems[b]).wait()
        pltpu.make_async_copy(q_hbm.at[sidx.at[b]], qs.at[b], gsems[b]).wait()
        pltpu.make_async_copy(v_hbm.at[sidx.at[b]], vs.at[b], gsems[b]).wait()

    def wait_scatter(b):
        pltpu.make_async_copy(
            vs.at[b], aggsh.at[didx.at[b]], ssems[b]).wait()

    def do_chunk(i, b):
        @pl.when(i >= 1)
        def _():
            wait_scatter(1 - b)

        @pl.when(i + 1 < NCHUNK)
        def _():
            issue_idx(i + 1, 1 - b)
            issue_gathers(1 - b)

        wait_gathers(b)

        def edge_one(e, c2):
            for j in range(H // 16):
                sl = pl.ds(j * 16, 16)
                kk = kd[b, e, sl]
                qq = qs[b, e, sl]
                vv = vs[b, e, sl]
                eta = 1.0 / (1.0 + jnp.exp(-(kk + qq)))
                vs[b, e, sl] = eta * vv
            return c2

        lax.fori_loop(0, C, edge_one, 0, unroll=2)
        pltpu.async_copy(vs.at[b], aggsh.at[didx.at[b]], ssems[b], add=True)

    issue_idx(0, 0)
    issue_gathers(0)

    def chunk_body(i, carry):
        @pl.when(lax.rem(i, 2) == 0)
        def _():
            do_chunk(i, 0)

        @pl.when(lax.rem(i, 2) == 1)
        def _():
            do_chunk(i, 1)

        return carry

    lax.fori_loop(0, NCHUNK, chunk_body, 0, unroll=False)
    wait_scatter((NCHUNK - 1) % 2)
    plsc.subcore_barrier()

    for c in range(NRCH):
        r = row0 + c * RCH
        pltpu.sync_copy(aggsh.at[pl.ds(r, RCH)], zbuf)
        pltpu.sync_copy(zbuf, out_hbm.at[cid, pl.ds(r, RCH)])


@jax.jit
def _edge_pass(k, q, v, src, dst, zeros):
    mesh = plsc.VectorSubcoreMesh(core_axis_name="c", subcore_axis_name="s")
    f = pl.kernel(
        _edge_body,
        out_type=jax.ShapeDtypeStruct((NC, NP, H), jnp.float32),
        mesh=mesh,
        scratch_types=[
            pltpu.VMEM((2, C), jnp.int32),
            pltpu.VMEM((2, C), jnp.int32),
            pltpu.VMEM((2, C, H), jnp.float32),
            pltpu.VMEM((2, C, H), jnp.float32),
            pltpu.VMEM((2, C, H), jnp.float32),
            [pltpu.SemaphoreType.DMA, pltpu.SemaphoreType.DMA],
            [pltpu.SemaphoreType.DMA, pltpu.SemaphoreType.DMA],
            pltpu.VMEM_SHARED((NP, H), jnp.float32),
        ],
    )
    return f(k, q, v, src, dst, zeros)


BLK = 2000


def _dense0_body(x_ref, wp_ref, bp_ref, wc_ref, bc_ref, out_ref):
    h = jnp.maximum(jnp.dot(x_ref[...], wp_ref[...],
                            preferred_element_type=jnp.float32)
                    + bp_ref[...], 0.0)
    out_ref[...] = jnp.dot(h, wc_ref[...],
                           preferred_element_type=jnp.float32) + bc_ref[...]


@jax.jit
def _dense0(x, wp, bp, wc, bc):
    m = wc.shape[1]
    return pl.pallas_call(
        _dense0_body,
        grid=(N // BLK,),
        in_specs=[
            pl.BlockSpec((BLK, H), lambda i: (i, 0)),
            pl.BlockSpec((H, H), lambda i: (0, 0)),
            pl.BlockSpec((1, H), lambda i: (0, 0)),
            pl.BlockSpec((H, m), lambda i: (0, 0)),
            pl.BlockSpec((1, m), lambda i: (0, 0)),
        ],
        out_specs=pl.BlockSpec((BLK, m), lambda i: (i, 0)),
        out_shape=jax.ShapeDtypeStruct((N, m), jnp.float32),
    )(x, wp, bp, wc, bc)


def _stats_body(a0_ref, a1_ref, s_ref, pre_ref, sum_ref, sq_ref):
    i = pl.program_id(0)
    pre = a0_ref[...] + a1_ref[...] + s_ref[...]
    pre_ref[...] = pre
    bs = jnp.sum(pre, axis=0, keepdims=True)
    bq = jnp.sum(pre * pre, axis=0, keepdims=True)

    @pl.when(i == 0)
    def _():
        sum_ref[...] = bs
        sq_ref[...] = bq

    @pl.when(i > 0)
    def _():
        sum_ref[...] += bs
        sq_ref[...] += bq


@jax.jit
def _stats(a0, a1, s):
    return pl.pallas_call(
        _stats_body,
        grid=(N // BLK,),
        in_specs=[pl.BlockSpec((BLK, H), lambda i: (i, 0))] * 3,
        out_specs=[
            pl.BlockSpec((BLK, H), lambda i: (i, 0)),
            pl.BlockSpec((1, H), lambda i: (0, 0)),
            pl.BlockSpec((1, H), lambda i: (0, 0)),
        ],
        out_shape=[
            jax.ShapeDtypeStruct((N, H), jnp.float32),
            jax.ShapeDtypeStruct((1, H), jnp.float32),
            jax.ShapeDtypeStruct((1, H), jnp.float32),
        ],
    )(a0, a1, s)


def _normproj_body(pre_ref, sum_ref, sq_ref, g_ref, be_ref, wc_ref, bc_ref,
                   out_ref):
    mu = sum_ref[...] / N
    var = sq_ref[...] / N - mu * mu
    scale = g_ref[...] * lax.rsqrt(var + 1e-5)
    h = jnp.maximum((pre_ref[...] - mu) * scale + be_ref[...], 0.0)
    out_ref[...] = jnp.dot(h, wc_ref[...],
                           preferred_element_type=jnp.float32) + bc_ref[...]


@jax.jit
def _normproj(pre, sm, sq, g, be, wc, bc):
    m = wc.shape[1]
    return pl.pallas_call(
        _normproj_body,
        grid=(N // BLK,),
        in_specs=[
            pl.BlockSpec((BLK, H), lambda i: (i, 0)),
            pl.BlockSpec((1, H), lambda i: (0, 0)),
            pl.BlockSpec((1, H), lambda i: (0, 0)),
            pl.BlockSpec((1, H), lambda i: (0, 0)),
            pl.BlockSpec((1, H), lambda i: (0, 0)),
            pl.BlockSpec((H, m), lambda i: (0, 0)),
            pl.BlockSpec((1, m), lambda i: (0, 0)),
        ],
        out_specs=pl.BlockSpec((BLK, m), lambda i: (i, 0)),
        out_shape=jax.ShapeDtypeStruct((N, m), jnp.float32),
    )(pre, sm, sq, g, be, wc, bc)


def _wcat(c):
    wc = jnp.concatenate([c['Wk'], c['Wq'], c['Wv'], c['Ws']], axis=1)
    bc = jnp.concatenate([c['bk'], c['bq'], c['bv'], c['b']])[None, :]
    return wc, bc


def kernel(x, ei, params):
    p = params
    zeros = jnp.zeros((RCH, H), jnp.float32)

    wc1, bc1 = _wcat(p['c1'])
    proj = _dense0(x, p['Wp'], p['bp'][None, :], wc1, bc1)

    for i in (1, 2, 3):
        k = proj[:, 0:H]
        q = proj[:, H:2 * H]
        v = proj[:, 2 * H:3 * H]
        s = proj[:, 3 * H:4 * H]
        aggp = _edge_pass(k, q, v, ei[0], ei[1], zeros)
        pre, sm, sq = _stats(aggp[0, :N], aggp[1, :N], s)
        if i < 3:
            wc, bc = _wcat(p['c%d' % (i + 1)])
        else:
            wc, bc = p['Wh'], p['bh'][None, :]
        proj = _normproj(pre, sm, sq, p['g%d' % i][None, :],
                         p['be%d' % i][None, :], wc, bc)
    return proj


# speedup vs baseline: 1.1217x; 1.1217x over previous
import jax
import jax.numpy as jnp
from jax import lax
from jax.experimental import pallas as pl
from jax.experimental.pallas import tpu as pltpu
from jax.experimental.pallas import tpu_sc as plsc

N = 10000
E = 320000
H = 128

NC = 2
NS = 16
NW = NC * NS
EPW = E // NW
C = 50
CPW = EPW // C
CPB = 8
NBLK = CPW // CPB
NP = 10240
RPT = NP // NS
RCH = 40
NRCH = RPT // RCH


def _edge_body(k_hbm, qv_hbm, src2_hbm, dst2_hbm, zeros_hbm, out_hbm,
               sidx, didx, kd, qvd, isems, gsems, ssems, aggsh):
    cid = lax.axis_index("c")
    sid = lax.axis_index("s")
    wid = sid * NC + cid

    row0 = sid * RPT
    for c in range(NRCH):
        pltpu.sync_copy(zeros_hbm, aggsh.at[pl.ds(row0 + c * RCH, RCH)])
    plsc.subcore_barrier()

    crow0 = wid * CPW

    def start_idx(nb, ib):
        base = crow0 + nb * CPB
        pltpu.async_copy(src2_hbm.at[pl.ds(base, CPB)], sidx.at[ib],
                         isems[ib])
        pltpu.async_copy(dst2_hbm.at[pl.ds(base, CPB)], didx.at[ib],
                         isems[ib])

    def wait_idx(ib):
        pltpu.make_async_copy(src2_hbm.at[pl.ds(crow0, CPB)], sidx.at[ib],
                              isems[ib]).wait()
        pltpu.make_async_copy(dst2_hbm.at[pl.ds(crow0, CPB)], didx.at[ib],
                              isems[ib]).wait()

    def start_gathers(ib, j, b):
        pltpu.async_copy(k_hbm.at[didx.at[ib, j]], kd.at[b], gsems[b])
        pltpu.async_copy(qv_hbm.at[sidx.at[ib, j]], qvd.at[b], gsems[b])

    def wait_gathers(ib, j, b):
        pltpu.make_async_copy(k_hbm.at[didx.at[ib, j]], kd.at[b],
                              gsems[b]).wait()
        pltpu.make_async_copy(qv_hbm.at[sidx.at[ib, j]], qvd.at[b],
                              gsems[b]).wait()

    def start_scatter(ib, j, b):
        pltpu.async_copy(kd.at[b], aggsh.at[didx.at[ib, j]], ssems[b],
                         add=True)

    def wait_scatter(ib, j, b):
        pltpu.make_async_copy(kd.at[b], aggsh.at[didx.at[ib, j]],
                              ssems[b]).wait()

    def do_chunk(nb, ib, j):
        b = j % 2

        if j > 0:
            wait_scatter(ib, j - 1, 1 - b)
        else:
            @pl.when(nb > 0)
            def _():
                wait_scatter(1 - ib, CPB - 1, 1 - b)

        if j + 1 < CPB:
            start_gathers(ib, j + 1, 1 - b)
        else:
            @pl.when(nb + 1 < NBLK)
            def _():
                wait_idx(1 - ib)
                start_gathers(1 - ib, 0, 1 - b)

        wait_gathers(ib, j, b)

        def edge_one(e, c2):
            for jj in range(H // 16):
                sl = pl.ds(jj * 16, 16)
                kk = kd[b, e, sl]
                qq = qvd[b, e, sl]
                vv = qvd[b, e, pl.ds(H + jj * 16, 16)]
                em = jnp.exp(-(kk + qq))
                kd[b, e, sl] = vv / (1.0 + em)
            return c2

        lax.fori_loop(0, C, edge_one, 0, unroll=False)
        start_scatter(ib, j, b)

    def run_block(nb, ib):
        do_chunk(nb, ib, 0)

        @pl.when(nb + 1 < NBLK)
        def _():
            start_idx(nb + 1, 1 - ib)

        for j in range(1, CPB):
            do_chunk(nb, ib, j)

    start_idx(0, 0)
    wait_idx(0)
    start_gathers(0, 0, 0)

    def block_body(nb, carry):
        @pl.when(lax.rem(nb, 2) == 0)
        def _():
            run_block(nb, 0)

        @pl.when(lax.rem(nb, 2) == 1)
        def _():
            run_block(nb, 1)

        return carry

    lax.fori_loop(0, NBLK, block_body, 0, unroll=False)
    wait_scatter(0, CPB - 1, 1)
    plsc.subcore_barrier()

    zbuf = kd.at[0, pl.ds(0, RCH)]
    for c in range(NRCH):
        r = row0 + c * RCH
        pltpu.sync_copy(aggsh.at[pl.ds(r, RCH)], zbuf)
        pltpu.sync_copy(zbuf, out_hbm.at[cid, pl.ds(r, RCH)])


@jax.jit
def _edge_pass(k, qv, src2, dst2, zeros):
    mesh = plsc.VectorSubcoreMesh(core_axis_name="c", subcore_axis_name="s")
    f = pl.kernel(
        _edge_body,
        out_type=jax.ShapeDtypeStruct((NC, NP, H), jnp.float32),
        mesh=mesh,
        scratch_types=[
            pltpu.VMEM((2, CPB, C), jnp.int32),
            pltpu.VMEM((2, CPB, C), jnp.int32),
            pltpu.VMEM((2, C, H), jnp.float32),
            pltpu.VMEM((2, C, 2 * H), jnp.float32),
            [pltpu.SemaphoreType.DMA, pltpu.SemaphoreType.DMA],
            [pltpu.SemaphoreType.DMA, pltpu.SemaphoreType.DMA],
            [pltpu.SemaphoreType.DMA, pltpu.SemaphoreType.DMA],
            pltpu.VMEM_SHARED((NP, H), jnp.float32),
        ],
    )
    return f(k, qv, src2, dst2, zeros)


BLK = 2000


def _dense0_body(x_ref, wp_ref, bp_ref, wc_ref, bc_ref,
                 k_ref, qv_ref, s_ref):
    h = jnp.maximum(jnp.dot(x_ref[...], wp_ref[...],
                            preferred_element_type=jnp.float32)
                    + bp_ref[...], 0.0)
    out = jnp.dot(h, wc_ref[...],
                  preferred_element_type=jnp.float32) + bc_ref[...]
    k_ref[...] = out[:, 0:H]
    qv_ref[...] = out[:, H:3 * H]
    s_ref[...] = out[:, 3 * H:4 * H]


@jax.jit
def _dense0(x, wp, bp, wc, bc):
    return pl.pallas_call(
        _dense0_body,
        grid=(N // BLK,),
        in_specs=[
            pl.BlockSpec((BLK, H), lambda i: (i, 0)),
            pl.BlockSpec((H, H), lambda i: (0, 0)),
            pl.BlockSpec((1, H), lambda i: (0, 0)),
            pl.BlockSpec((H, 4 * H), lambda i: (0, 0)),
            pl.BlockSpec((1, 4 * H), lambda i: (0, 0)),
        ],
        out_specs=[
            pl.BlockSpec((BLK, H), lambda i: (i, 0)),
            pl.BlockSpec((BLK, 2 * H), lambda i: (i, 0)),
            pl.BlockSpec((BLK, H), lambda i: (i, 0)),
        ],
        out_shape=[
            jax.ShapeDtypeStruct((N, H), jnp.float32),
            jax.ShapeDtypeStruct((N, 2 * H), jnp.float32),
            jax.ShapeDtypeStruct((N, H), jnp.float32),
        ],
    )(x, wp, bp, wc, bc)


def _stats_body(a0_ref, a1_ref, s_ref, pre_ref, sum_ref, sq_ref):
    i = pl.program_id(0)
    pre = a0_ref[...] + a1_ref[...] + s_ref[...]
    pre_ref[...] = pre
    bs = jnp.sum(pre, axis=0, keepdims=True)
    bq = jnp.sum(pre * pre, axis=0, keepdims=True)

    @pl.when(i == 0)
    def _():
        sum_ref[...] = bs
        sq_ref[...] = bq

    @pl.when(i > 0)
    def _():
        sum_ref[...] += bs
        sq_ref[...] += bq


@jax.jit
def _stats(a0, a1, s):
    return pl.pallas_call(
        _stats_body,
        grid=(N // BLK,),
        in_specs=[pl.BlockSpec((BLK, H), lambda i: (i, 0))] * 3,
        out_specs=[
            pl.BlockSpec((BLK, H), lambda i: (i, 0)),
            pl.BlockSpec((1, H), lambda i: (0, 0)),
            pl.BlockSpec((1, H), lambda i: (0, 0)),
        ],
        out_shape=[
            jax.ShapeDtypeStruct((N, H), jnp.float32),
            jax.ShapeDtypeStruct((1, H), jnp.float32),
            jax.ShapeDtypeStruct((1, H), jnp.float32),
        ],
    )(a0, a1, s)


def _normproj_body(pre_ref, sum_ref, sq_ref, g_ref, be_ref, wc_ref, bc_ref,
                   k_ref, qv_ref, s_ref):
    mu = sum_ref[...] / N
    var = sq_ref[...] / N - mu * mu
    scale = g_ref[...] * lax.rsqrt(var + 1e-5)
    h = jnp.maximum((pre_ref[...] - mu) * scale + be_ref[...], 0.0)
    out = jnp.dot(h, wc_ref[...],
                  preferred_element_type=jnp.float32) + bc_ref[...]
    k_ref[...] = out[:, 0:H]
    qv_ref[...] = out[:, H:3 * H]
    s_ref[...] = out[:, 3 * H:4 * H]


@jax.jit
def _normproj(pre, sm, sq, g, be, wc, bc):
    return pl.pallas_call(
        _normproj_body,
        grid=(N // BLK,),
        in_specs=[
            pl.BlockSpec((BLK, H), lambda i: (i, 0)),
            pl.BlockSpec((1, H), lambda i: (0, 0)),
            pl.BlockSpec((1, H), lambda i: (0, 0)),
            pl.BlockSpec((1, H), lambda i: (0, 0)),
            pl.BlockSpec((1, H), lambda i: (0, 0)),
            pl.BlockSpec((H, 4 * H), lambda i: (0, 0)),
            pl.BlockSpec((1, 4 * H), lambda i: (0, 0)),
        ],
        out_specs=[
            pl.BlockSpec((BLK, H), lambda i: (i, 0)),
            pl.BlockSpec((BLK, 2 * H), lambda i: (i, 0)),
            pl.BlockSpec((BLK, H), lambda i: (i, 0)),
        ],
        out_shape=[
            jax.ShapeDtypeStruct((N, H), jnp.float32),
            jax.ShapeDtypeStruct((N, 2 * H), jnp.float32),
            jax.ShapeDtypeStruct((N, H), jnp.float32),
        ],
    )(pre, sm, sq, g, be, wc, bc)


def _head_body(pre_ref, sum_ref, sq_ref, g_ref, be_ref, wc_ref, bc_ref,
               out_ref):
    mu = sum_ref[...] / N
    var = sq_ref[...] / N - mu * mu
    scale = g_ref[...] * lax.rsqrt(var + 1e-5)
    h = jnp.maximum((pre_ref[...] - mu) * scale + be_ref[...], 0.0)
    out_ref[...] = jnp.dot(h, wc_ref[...],
                           preferred_element_type=jnp.float32) + bc_ref[...]


@jax.jit
def _head(pre, sm, sq, g, be, wc, bc):
    m = wc.shape[1]
    return pl.pallas_call(
        _head_body,
        grid=(N // BLK,),
        in_specs=[
            pl.BlockSpec((BLK, H), lambda i: (i, 0)),
            pl.BlockSpec((1, H), lambda i: (0, 0)),
            pl.BlockSpec((1, H), lambda i: (0, 0)),
            pl.BlockSpec((1, H), lambda i: (0, 0)),
            pl.BlockSpec((1, H), lambda i: (0, 0)),
            pl.BlockSpec((H, m), lambda i: (0, 0)),
            pl.BlockSpec((1, m), lambda i: (0, 0)),
        ],
        out_specs=pl.BlockSpec((BLK, m), lambda i: (i, 0)),
        out_shape=jax.ShapeDtypeStruct((N, m), jnp.float32),
    )(pre, sm, sq, g, be, wc, bc)


def _wcat(c):
    wc = jnp.concatenate([c['Wk'], c['Wq'], c['Wv'], c['Ws']], axis=1)
    bc = jnp.concatenate([c['bk'], c['bq'], c['bv'], c['b']])[None, :]
    return wc, bc


def kernel(x, ei, params):
    p = params
    zeros = jnp.zeros((RCH, H), jnp.float32)
    src2 = ei[0].reshape(E // C, C)
    dst2 = ei[1].reshape(E // C, C)

    wc1, bc1 = _wcat(p['c1'])
    k, qv, s = _dense0(x, p['Wp'], p['bp'][None, :], wc1, bc1)

    for i in (1, 2, 3):
        aggp = _edge_pass(k, qv, src2, dst2, zeros)
        pre, sm, sq = _stats(aggp[0, :N], aggp[1, :N], s)
        g = p['g%d' % i][None, :]
        be = p['be%d' % i][None, :]
        if i < 3:
            wc, bc = _wcat(p['c%d' % (i + 1)])
            k, qv, s = _normproj(pre, sm, sq, g, be, wc, bc)
        else:
            out = _head(pre, sm, sq, g, be, p['Wh'], p['bh'][None, :])
    return out
